# Initial kernel scaffold; baseline (speedup 1.0000x reference)
import jax
import jax.numpy as jnp
from jax.experimental import pallas as pl


def kernel(x, edge_index, params):
    raise NotImplementedError("write your pallas kernel here")


# speedup vs baseline: 4.6297x; 4.6297x over previous
import functools

import jax
import jax.numpy as jnp
from jax import lax
from jax.experimental import pallas as pl
from jax.experimental.pallas import tpu as pltpu
from jax.experimental.pallas import tpu_sc as plsc

N = 10000
E = 160000
HID = 256
HEADS = 4
FW = HID * HEADS

NC = 2
NS = 16
NW = NC * NS
LANES = 16

EPT = 5008
E_PAD = NW * EPT
E_ALLOC = E_PAD + 2048

R_CHUNK = 80
D_SC = 64
SC_HALF = R_CHUNK * D_SC
N_PAD = NC * SC_HALF
BLK = 512
NEG = -3.0e38

_mesh = plsc.VectorSubcoreMesh(core_axis_name="c", subcore_axis_name="s")
_sc_params = pltpu.CompilerParams(use_tc_tiling_on_sc=False,
                                  needs_layout_passes=False)


def _mm(a, w, b2d, c=None, relu=False):
    m, kdim = a.shape
    dout = w.shape[1]
    mt = 400
    nt = 256 if dout % 256 == 0 else 128

    def body(*refs):
        if c is None:
            a_ref, w_ref, b_ref, o_ref = refs
            acc = jnp.dot(a_ref[...], w_ref[...],
                          preferred_element_type=jnp.float32)
            acc = acc + b_ref[0][None, :]
        else:
            a_ref, w_ref, b_ref, c_ref, o_ref = refs
            acc = jnp.dot(a_ref[...], w_ref[...],
                          preferred_element_type=jnp.float32)
            acc = acc + b_ref[0][None, :] + c_ref[...]
        if relu:
            acc = jnp.maximum(acc, 0.0)
        o_ref[...] = acc

    in_specs = [
        pl.BlockSpec((mt, kdim), lambda i, j: (i, 0)),
        pl.BlockSpec((kdim, nt), lambda i, j: (0, j)),
        pl.BlockSpec((8, nt), lambda i, j: (0, j)),
    ]
    args = [a, w, b2d]
    if c is not None:
        in_specs.append(pl.BlockSpec((mt, nt), lambda i, j: (i, j)))
        args.append(c)
    return pl.pallas_call(
        body,
        grid=(m // mt, dout // nt),
        in_specs=in_specs,
        out_specs=pl.BlockSpec((mt, nt), lambda i, j: (i, j)),
        out_shape=jax.ShapeDtypeStruct((m, dout), jnp.float32),
    )(*args)


@functools.partial(
    pl.kernel,
    out_type=[
        jax.ShapeDtypeStruct((HEADS * E_ALLOC,), jnp.float32),
        jax.ShapeDtypeStruct((NW * HEADS * LANES,), jnp.float32),
    ],
    mesh=_mesh,
    scratch_types=[
        pltpu.VMEM((EPT,), jnp.int32),
        pltpu.VMEM((EPT,), jnp.int32),
        pltpu.VMEM((HEADS, EPT), jnp.float32),
        pltpu.VMEM((LANES, FW), jnp.float32),
        pltpu.VMEM((LANES, FW), jnp.float32),
        pltpu.VMEM((LANES,), jnp.int32),
        pltpu.VMEM((LANES,), jnp.int32),
        pltpu.VMEM((HEADS * LANES,), jnp.float32),
        pltpu.SemaphoreType.DMA,
        pltpu.SemaphoreType.DMA,
    ],
    compiler_params=_sc_params,
)
def _phase_a(q_hbm, k_hbm, src_hbm, dst_hbm, alpha_hbm, tmax_hbm,
             dst_t, src_t, alpha_st, qbuf, kbuf, qidx, kidx, rm,
             sem_q, sem_k):
    cid = lax.axis_index("c")
    sid = lax.axis_index("s")
    wid = sid * NC + cid
    base = pl.multiple_of(wid * EPT, 8)

    pltpu.sync_copy(dst_hbm.at[pl.ds(base, EPT)], dst_t)
    pltpu.sync_copy(src_hbm.at[pl.ds(base, EPT)], src_t)

    neg = jnp.full((LANES,), NEG, jnp.float32)
    for h in range(HEADS):
        rm[pl.ds(h * LANES, LANES)] = neg

    iota = lax.iota(jnp.int32, LANES)
    zero16 = jnp.zeros((LANES,), jnp.float32)

    def group(g, carry):
        qidx[...] = dst_t[pl.ds(g * LANES, LANES)]
        kidx[...] = src_t[pl.ds(g * LANES, LANES)]
        cq = pltpu.async_copy(q_hbm.at[qidx], qbuf, sem_q)
        ck = pltpu.async_copy(k_hbm.at[kidx], kbuf, sem_k)
        cq.wait()
        ck.wait()

        def edge_dot(r, als):
            lane = iota == r
            new = []
            for h in range(HEADS):
                acc = (qbuf[r, pl.ds(h * HID, LANES)]
                       * kbuf[r, pl.ds(h * HID, LANES)])
                for j in range(1, HID // LANES):
                    col = h * HID + j * LANES
                    acc = acc + (qbuf[r, pl.ds(col, LANES)]
                                 * kbuf[r, pl.ds(col, LANES)])
                s = jnp.sum(acc) * 0.0625
                new.append(jnp.where(lane, s, als[h]))
            return tuple(new)

        als = lax.fori_loop(0, LANES, edge_dot, (zero16,) * HEADS)
        for h in range(HEADS):
            alpha_st[h, pl.ds(g * LANES, LANES)] = als[h]
            rm[pl.ds(h * LANES, LANES)] = jnp.maximum(
                rm[pl.ds(h * LANES, LANES)], als[h])
        return carry

    lax.fori_loop(0, EPT // LANES, group, 0)

    for h in range(HEADS):
        pltpu.sync_copy(alpha_st.at[h],
                        alpha_hbm.at[pl.ds(pl.multiple_of(h * E_ALLOC + base, 8), EPT)])
    pltpu.sync_copy(rm, tmax_hbm.at[pl.ds(pl.multiple_of(wid * HEADS * LANES, 8), HEADS * LANES)])


@functools.partial(
    pl.kernel,
    out_type=jax.ShapeDtypeStruct((N_PAD, FW), jnp.float32),
    mesh=_mesh,
    scratch_types=[
        pltpu.VMEM((R_CHUNK, FW), jnp.float32),
        pltpu.VMEM((LANES, FW), jnp.float32),
        pltpu.VMEM((BLK,), jnp.int32),
        pltpu.VMEM((BLK,), jnp.int32),
        pltpu.VMEM((HEADS, BLK), jnp.float32),
        pltpu.VMEM((128, LANES), jnp.float32),
        pltpu.VMEM((128,), jnp.int32),
        pltpu.VMEM((LANES,), jnp.int32),
        pltpu.VMEM((R_CHUNK, LANES), jnp.float32),
        pltpu.VMEM((NW * HEADS * LANES,), jnp.float32),
        pltpu.VMEM((32,), jnp.int32),
        pltpu.VMEM((144,), jnp.int32),
        pltpu.VMEM_SHARED((SC_HALF, LANES), jnp.float32),
        pltpu.SemaphoreType.DMA,
    ],
    compiler_params=_sc_params,
)
def _phase_bc(v_hbm, src_hbm, dst_hbm, alpha_hbm, tmax_hbm, scb_hbm, cb_hbm,
              out_hbm,
              acc, vbuf, dst_blk, src_blk, alpha_blk, exrow, exidx, vidx,
              den_local, tmax_v, scb_v, cb_v, dshared, sem):
    cid = lax.axis_index("c")
    sid = lax.axis_index("s")

    pltpu.sync_copy(tmax_hbm, tmax_v)
    pltpu.sync_copy(scb_hbm, scb_v)
    pltpu.sync_copy(cb_hbm, cb_v)

    gm = []
    for h in range(HEADS):
        m = tmax_v[pl.ds(h * LANES, LANES)]
        for w in range(1, NW):
            m = jnp.maximum(
                m, tmax_v[pl.ds((w * HEADS + h) * LANES, LANES)])
        gm.append(jnp.broadcast_to(jnp.max(m), (LANES,)))

    iota = lax.iota(jnp.int32, LANES)
    zero16 = jnp.zeros((LANES,), jnp.float32)

    def _sread(ref, idx):
        return ref[pl.ds(idx, LANES)][0]

    for r in range(R_CHUNK):
        den_local[r, :] = zero16
    @pl.when(sid == 0)
    def _zero_shared():
        def zs(i, carry):
            pltpu.sync_copy(den_local, dshared.at[pl.ds(i * R_CHUNK, R_CHUNK)])
            return carry
        lax.fori_loop(0, D_SC, zs, 0)
    plsc.subcore_barrier()

    for r in range(128):
        exrow[r, :] = zero16

    lo_sc = _sread(scb_v, cid)
    hi_sc = _sread(scb_v, cid + 1)
    per = (hi_sc - lo_sc + NS - 1) >> 4
    mylo = lo_sc + sid * per
    myhi = jnp.minimum(mylo + per, hi_sc)
    blk0 = mylo & ~7
    nblk = jnp.maximum(0, (myhi - blk0 + BLK - 1) >> 9)
    dbase = cid * SC_HALF

    def p1_block(b, carry):
        estart = pl.multiple_of(blk0 + b * BLK, 8)
        pltpu.sync_copy(dst_hbm.at[pl.ds(estart, BLK)], dst_blk)
        for h in range(HEADS):
            pltpu.sync_copy(alpha_hbm.at[pl.ds(pl.multiple_of(h * E_ALLOC + estart, 8), BLK)],
                            alpha_blk.at[h])

        def p1_group(g, carry2):
            gb = lax.rem(g, 8)
            dstv = dst_blk[pl.ds(g * LANES, LANES)]
            eglob = jnp.broadcast_to(estart + g * LANES, (LANES,)) + iota
            valid = (eglob >= mylo) & (eglob < myhi)
            idxv = jnp.where(valid, dstv - dbase, 0)
            exs = []
            for h in range(HEADS):
                av = alpha_blk[h, pl.ds(g * LANES, LANES)]
                ex = jnp.exp(av - gm[h])
                exs.append(jnp.where(valid, ex, 0.0))
            gbase = gb * LANES
            for r in range(LANES):
                rowv = jnp.where(iota == 0, exs[0][r], 0.0)
                for h in range(1, HEADS):
                    rowv = jnp.where(iota == h, exs[h][r], rowv)
                exrow[gbase + r, :] = rowv
            exidx[pl.ds(gbase, LANES)] = idxv

            @pl.when(gb == 7)
            def _flush():
                pltpu.sync_copy(exrow, dshared.at[exidx], add=True)
            return carry2

        lax.fori_loop(0, BLK // LANES, p1_group, 0)
        return carry

    lax.fori_loop(0, nblk, p1_block, 0)
    plsc.subcore_barrier()

    def chunk_body(i, carry0):
        c_sc = sid + NS * i
        c_glob = cid * D_SC + c_sc
        clo = c_glob * R_CHUNK

        pltpu.sync_copy(dshared.at[pl.ds(c_sc * R_CHUNK, R_CHUNK)], den_local)

        def zacc(r, carry):
            for j in range(FW // LANES):
                acc[r, pl.ds(j * LANES, LANES)] = zero16
            return carry
        lax.fori_loop(0, R_CHUNK, zacc, 0)

        lo = _sread(cb_v, c_glob)
        hi = _sread(cb_v, c_glob + 1)
        cblk0 = lo & ~7
        cnblk = jnp.maximum(0, (hi - cblk0 + BLK - 1) >> 9)

        def p2_block(b, carry):
            estart = pl.multiple_of(cblk0 + b * BLK, 8)
            pltpu.sync_copy(dst_hbm.at[pl.ds(estart, BLK)], dst_blk)
            pltpu.sync_copy(src_hbm.at[pl.ds(estart, BLK)], src_blk)
            for h in range(HEADS):
                pltpu.sync_copy(alpha_hbm.at[pl.ds(pl.multiple_of(h * E_ALLOC + estart, 8), BLK)],
                                alpha_blk.at[h])

            def p2_group(g, carry2):
                gl0 = estart + g * LANES

                @pl.when((gl0 < hi) & (gl0 + LANES - 1 >= lo))
                def _do_group():
                    vidx[...] = src_blk[pl.ds(g * LANES, LANES)]
                    pltpu.async_copy(v_hbm.at[vidx], vbuf, sem).wait()
                    dstv = dst_blk[pl.ds(g * LANES, LANES)]
                    eglobv = jnp.broadcast_to(gl0, (LANES,)) + iota
                    okv = (eglobv >= lo) & (eglobv < hi)
                    rowlv = jnp.where(okv, dstv - clo, 0)
                    ams = []
                    for h in range(HEADS):
                        exv = jnp.exp(
                            alpha_blk[h, pl.ds(g * LANES, LANES)] - gm[h])
                        denv = plsc.load_gather(
                            den_local,
                            [rowlv, jnp.full((LANES,), h, jnp.int32)])
                        ams.append(jnp.where(okv, exv / denv, 0.0))
                    for r in range(LANES):
                        rowl = rowlv[r]
                        for h in range(HEADS):
                            a_sp = jnp.broadcast_to(ams[h][r], (LANES,))
                            for jj in range(HID // LANES):
                                col = h * HID + jj * LANES
                                plsc.addupdate(
                                    acc.at[rowl, pl.ds(col, LANES)],
                                    vbuf[r, pl.ds(col, LANES)] * a_sp)
                return carry2

            lax.fori_loop(0, BLK // LANES, p2_group, 0)
            return carry

        lax.fori_loop(0, cnblk, p2_block, 0)
        pltpu.sync_copy(acc, out_hbm.at[pl.ds(clo, R_CHUNK)])
        return carry0

    lax.fori_loop(0, D_SC // NS, chunk_body, 0)


def kernel(x, edge_index, params):
    src = edge_index[0].astype(jnp.int32)
    dst = edge_index[1].astype(jnp.int32)
    order = jnp.argsort(dst)
    dsts = dst[order]
    srcs = src[order]
    padlen = E_ALLOC - E
    dsts_p = jnp.concatenate([dsts, jnp.zeros((padlen,), jnp.int32)])
    srcs_p = jnp.concatenate([srcs, jnp.zeros((padlen,), jnp.int32)])

    mid = jnp.searchsorted(dsts, SC_HALF).astype(jnp.int32)
    scb = jnp.concatenate([
        jnp.zeros((1,), jnp.int32),
        mid[None],
        jnp.full((30,), E, jnp.int32),
    ])
    cgrid = jnp.arange(0, N_PAD + 1, R_CHUNK, dtype=jnp.int32)
    cb = jnp.searchsorted(dsts, cgrid).astype(jnp.int32)
    cb = jnp.concatenate([cb, jnp.full((144 - cb.shape[0],), E, jnp.int32)])

    b2 = lambda b: jnp.tile(b[None, :], (8, 1))

    h = _mm(x, params["proj_W"], b2(params["proj_b"]))
    for i in range(3):
        lp = params["layers"][i]
        q = _mm(h, lp["Wq"], b2(lp["bq"]))
        k = _mm(h, lp["Wk"], b2(lp["bk"]))
        v = _mm(h, lp["Wv"], b2(lp["bv"]))
        alpha, tmax = _phase_a(q, k, srcs_p, dsts_p)
        attn = _phase_bc(v, srcs_p, dsts_p, alpha, tmax, scb, cb)
        h = _mm(h, lp["Ws"], b2(lp["bs"]), c=attn[:N], relu=(i < 2))

    g1 = _mm(h, params["cls_W1"], b2(params["cls_b1"]), relu=True)
    w2 = jnp.pad(params["cls_W2"], ((0, 0), (0, 7)))
    bias2 = jnp.pad(params["cls_b2"], (0, 7))
    out = _mm(g1, w2, b2(bias2))
    return out[:, :121]


# speedup vs baseline: 5.0701x; 1.0951x over previous
import functools

import jax
import jax.numpy as jnp
from jax import lax
from jax.experimental import pallas as pl
from jax.experimental.pallas import tpu as pltpu
from jax.experimental.pallas import tpu_sc as plsc

N = 10000
E = 160000
HID = 256
HEADS = 4
FW = HID * HEADS

NC = 2
NS = 16
NW = NC * NS
LANES = 16

EPT = 5008
E_PAD = NW * EPT
E_ALLOC = E_PAD + 2048

R_CHUNK = 80
D_SC = 64
SC_HALF = R_CHUNK * D_SC
N_PAD = NC * SC_HALF
BLK = 512
NEG = -3.0e38

_mesh = plsc.VectorSubcoreMesh(core_axis_name="c", subcore_axis_name="s")
_sc_params = pltpu.CompilerParams(use_tc_tiling_on_sc=False,
                                  needs_layout_passes=False)


def _mm(a, w, b2d, c=None, relu=False):
    m, kdim = a.shape
    dout = w.shape[1]
    mt = 400
    nt = 256 if dout % 256 == 0 else 128

    def body(*refs):
        if c is None:
            a_ref, w_ref, b_ref, o_ref = refs
            acc = jnp.dot(a_ref[...], w_ref[...],
                          preferred_element_type=jnp.float32)
            acc = acc + b_ref[0][None, :]
        else:
            a_ref, w_ref, b_ref, c_ref, o_ref = refs
            acc = jnp.dot(a_ref[...], w_ref[...],
                          preferred_element_type=jnp.float32)
            acc = acc + b_ref[0][None, :] + c_ref[...]
        if relu:
            acc = jnp.maximum(acc, 0.0)
        o_ref[...] = acc

    in_specs = [
        pl.BlockSpec((mt, kdim), lambda i, j: (i, 0)),
        pl.BlockSpec((kdim, nt), lambda i, j: (0, j)),
        pl.BlockSpec((8, nt), lambda i, j: (0, j)),
    ]
    args = [a, w, b2d]
    if c is not None:
        in_specs.append(pl.BlockSpec((mt, nt), lambda i, j: (i, j)))
        args.append(c)
    return pl.pallas_call(
        body,
        grid=(m // mt, dout // nt),
        in_specs=in_specs,
        out_specs=pl.BlockSpec((mt, nt), lambda i, j: (i, j)),
        out_shape=jax.ShapeDtypeStruct((m, dout), jnp.float32),
    )(*args)


@functools.partial(
    pl.kernel,
    out_type=[
        jax.ShapeDtypeStruct((HEADS * E_ALLOC,), jnp.float32),
        jax.ShapeDtypeStruct((NW * HEADS * LANES,), jnp.float32),
    ],
    mesh=_mesh,
    scratch_types=[
        pltpu.VMEM((EPT,), jnp.int32),
        pltpu.VMEM((EPT,), jnp.int32),
        pltpu.VMEM((HEADS, EPT), jnp.float32),
        pltpu.VMEM((LANES, FW), jnp.float32),
        pltpu.VMEM((LANES, FW), jnp.float32),
        pltpu.VMEM((LANES, FW), jnp.float32),
        pltpu.VMEM((LANES, FW), jnp.float32),
        pltpu.VMEM((LANES,), jnp.int32),
        pltpu.VMEM((LANES,), jnp.int32),
        pltpu.VMEM((LANES,), jnp.int32),
        pltpu.VMEM((LANES,), jnp.int32),
        pltpu.VMEM((HEADS * LANES,), jnp.float32),
        pltpu.SemaphoreType.DMA,
        pltpu.SemaphoreType.DMA,
        pltpu.SemaphoreType.DMA,
        pltpu.SemaphoreType.DMA,
    ],
    compiler_params=_sc_params,
)
def _phase_a(q_hbm, k_hbm, src_hbm, dst_hbm, alpha_hbm, tmax_hbm,
             dst_t, src_t, alpha_st, qbuf, kbuf, qbuf1, kbuf1,
             qidx, kidx, qidx1, kidx1, rm,
             sem_q, sem_k, sem_q1, sem_k1):
    cid = lax.axis_index("c")
    sid = lax.axis_index("s")
    wid = sid * NC + cid
    base = pl.multiple_of(wid * EPT, 8)

    pltpu.sync_copy(dst_hbm.at[pl.ds(base, EPT)], dst_t)
    pltpu.sync_copy(src_hbm.at[pl.ds(base, EPT)], src_t)

    neg = jnp.full((LANES,), NEG, jnp.float32)
    for h in range(HEADS):
        rm[pl.ds(h * LANES, LANES)] = neg

    iota = lax.iota(jnp.int32, LANES)
    zero16 = jnp.zeros((LANES,), jnp.float32)

    def fire(g, qi, ki, qb, kb, sq, sk):
        qi[...] = dst_t[pl.ds(g * LANES, LANES)]
        ki[...] = src_t[pl.ds(g * LANES, LANES)]
        pltpu.async_copy(q_hbm.at[qi], qb, sq)
        pltpu.async_copy(k_hbm.at[ki], kb, sk)

    def waitbuf(qi, ki, qb, kb, sq, sk):
        pltpu.make_async_copy(q_hbm.at[qi], qb, sq).wait()
        pltpu.make_async_copy(k_hbm.at[ki], kb, sk).wait()

    def compute(g, qb, kb):
        def edge_dot(r, als):
            lane = iota == r
            new = []
            for h in range(HEADS):
                acc = (qb[r, pl.ds(h * HID, LANES)]
                       * kb[r, pl.ds(h * HID, LANES)])
                for j in range(1, HID // LANES):
                    col = h * HID + j * LANES
                    acc = acc + (qb[r, pl.ds(col, LANES)]
                                 * kb[r, pl.ds(col, LANES)])
                s = jnp.sum(acc) * 0.0625
                new.append(jnp.where(lane, s, als[h]))
            return tuple(new)

        als = lax.fori_loop(0, LANES, edge_dot, (zero16,) * HEADS)
        for h in range(HEADS):
            alpha_st[h, pl.ds(g * LANES, LANES)] = als[h]
            rm[pl.ds(h * LANES, LANES)] = jnp.maximum(
                rm[pl.ds(h * LANES, LANES)], als[h])

    ngroups = EPT // LANES
    fire(0, qidx, kidx, qbuf, kbuf, sem_q, sem_k)

    def pair(p, carry):
        g = p * 2
        fire(g + 1, qidx1, kidx1, qbuf1, kbuf1, sem_q1, sem_k1)
        waitbuf(qidx, kidx, qbuf, kbuf, sem_q, sem_k)
        compute(g, qbuf, kbuf)
        fire(g + 2, qidx, kidx, qbuf, kbuf, sem_q, sem_k)
        waitbuf(qidx1, kidx1, qbuf1, kbuf1, sem_q1, sem_k1)
        compute(g + 1, qbuf1, kbuf1)
        return carry

    lax.fori_loop(0, ngroups // 2, pair, 0)
    waitbuf(qidx, kidx, qbuf, kbuf, sem_q, sem_k)
    compute(ngroups - 1, qbuf, kbuf)

    for h in range(HEADS):
        pltpu.sync_copy(alpha_st.at[h],
                        alpha_hbm.at[pl.ds(pl.multiple_of(h * E_ALLOC + base, 8), EPT)])
    pltpu.sync_copy(rm, tmax_hbm.at[pl.ds(pl.multiple_of(wid * HEADS * LANES, 8), HEADS * LANES)])


@functools.partial(
    pl.kernel,
    out_type=jax.ShapeDtypeStruct((N_PAD, FW), jnp.float32),
    mesh=_mesh,
    scratch_types=[
        pltpu.VMEM((R_CHUNK, FW), jnp.float32),
        pltpu.VMEM((LANES, FW), jnp.float32),
        pltpu.VMEM((BLK,), jnp.int32),
        pltpu.VMEM((BLK,), jnp.int32),
        pltpu.VMEM((HEADS, BLK), jnp.float32),
        pltpu.VMEM((128, LANES), jnp.float32),
        pltpu.VMEM((128,), jnp.int32),
        pltpu.VMEM((LANES,), jnp.int32),
        pltpu.VMEM((R_CHUNK, LANES), jnp.float32),
        pltpu.VMEM((NW * HEADS * LANES,), jnp.float32),
        pltpu.VMEM((32,), jnp.int32),
        pltpu.VMEM((144,), jnp.int32),
        pltpu.VMEM_SHARED((SC_HALF, LANES), jnp.float32),
        pltpu.SemaphoreType.DMA,
    ],
    compiler_params=_sc_params,
)
def _phase_bc(v_hbm, src_hbm, dst_hbm, alpha_hbm, tmax_hbm, scb_hbm, cb_hbm,
              out_hbm,
              acc, vbuf, dst_blk, src_blk, alpha_blk, exrow, exidx, vidx,
              den_local, tmax_v, scb_v, cb_v, dshared, sem):
    cid = lax.axis_index("c")
    sid = lax.axis_index("s")

    pltpu.sync_copy(tmax_hbm, tmax_v)
    pltpu.sync_copy(scb_hbm, scb_v)
    pltpu.sync_copy(cb_hbm, cb_v)

    gm = []
    for h in range(HEADS):
        m = tmax_v[pl.ds(h * LANES, LANES)]
        for w in range(1, NW):
            m = jnp.maximum(
                m, tmax_v[pl.ds((w * HEADS + h) * LANES, LANES)])
        gm.append(jnp.broadcast_to(jnp.max(m), (LANES,)))

    iota = lax.iota(jnp.int32, LANES)
    zero16 = jnp.zeros((LANES,), jnp.float32)

    def _sread(ref, idx):
        return ref[pl.ds(idx, LANES)][0]

    for r in range(R_CHUNK):
        den_local[r, :] = zero16
    @pl.when(sid == 0)
    def _zero_shared():
        def zs(i, carry):
            pltpu.sync_copy(den_local, dshared.at[pl.ds(i * R_CHUNK, R_CHUNK)])
            return carry
        lax.fori_loop(0, D_SC, zs, 0)
    plsc.subcore_barrier()

    for r in range(128):
        exrow[r, :] = zero16

    lo_sc = _sread(scb_v, cid)
    hi_sc = _sread(scb_v, cid + 1)
    per = (hi_sc - lo_sc + NS - 1) >> 4
    mylo = lo_sc + sid * per
    myhi = jnp.minimum(mylo + per, hi_sc)
    blk0 = mylo & ~7
    nblk = jnp.maximum(0, (myhi - blk0 + BLK - 1) >> 9)
    dbase = cid * SC_HALF

    def p1_block(b, carry):
        estart = pl.multiple_of(blk0 + b * BLK, 8)
        pltpu.sync_copy(dst_hbm.at[pl.ds(estart, BLK)], dst_blk)
        for h in range(HEADS):
            pltpu.sync_copy(alpha_hbm.at[pl.ds(pl.multiple_of(h * E_ALLOC + estart, 8), BLK)],
                            alpha_blk.at[h])

        def p1_group(g, carry2):
            gb = lax.rem(g, 8)
            dstv = dst_blk[pl.ds(g * LANES, LANES)]
            eglob = jnp.broadcast_to(estart + g * LANES, (LANES,)) + iota
            valid = (eglob >= mylo) & (eglob < myhi)
            idxv = jnp.where(valid, dstv - dbase, 0)
            exs = []
            for h in range(HEADS):
                av = alpha_blk[h, pl.ds(g * LANES, LANES)]
                ex = jnp.exp(av - gm[h])
                exs.append(jnp.where(valid, ex, 0.0))
            gbase = gb * LANES
            for r in range(LANES):
                rowv = jnp.where(iota == 0, exs[0][r], 0.0)
                for h in range(1, HEADS):
                    rowv = jnp.where(iota == h, exs[h][r], rowv)
                exrow[gbase + r, :] = rowv
            exidx[pl.ds(gbase, LANES)] = idxv

            @pl.when(gb == 7)
            def _flush():
                pltpu.sync_copy(exrow, dshared.at[exidx], add=True)
            return carry2

        lax.fori_loop(0, BLK // LANES, p1_group, 0)
        return carry

    lax.fori_loop(0, nblk, p1_block, 0)
    plsc.subcore_barrier()

    def chunk_body(i, carry0):
        c_sc = sid + NS * i
        c_glob = cid * D_SC + c_sc
        clo = c_glob * R_CHUNK

        pltpu.sync_copy(dshared.at[pl.ds(c_sc * R_CHUNK, R_CHUNK)], den_local)

        def zacc(r, carry):
            for j in range(FW // LANES):
                acc[r, pl.ds(j * LANES, LANES)] = zero16
            return carry
        lax.fori_loop(0, R_CHUNK, zacc, 0)

        lo = _sread(cb_v, c_glob)
        hi = _sread(cb_v, c_glob + 1)
        cblk0 = lo & ~7
        cnblk = jnp.maximum(0, (hi - cblk0 + BLK - 1) >> 9)

        def p2_block(b, carry):
            estart = pl.multiple_of(cblk0 + b * BLK, 8)
            pltpu.sync_copy(dst_hbm.at[pl.ds(estart, BLK)], dst_blk)
            pltpu.sync_copy(src_hbm.at[pl.ds(estart, BLK)], src_blk)
            for h in range(HEADS):
                pltpu.sync_copy(alpha_hbm.at[pl.ds(pl.multiple_of(h * E_ALLOC + estart, 8), BLK)],
                                alpha_blk.at[h])

            def p2_group(g, carry2):
                gl0 = estart + g * LANES

                @pl.when((gl0 < hi) & (gl0 + LANES - 1 >= lo))
                def _do_group():
                    vidx[...] = src_blk[pl.ds(g * LANES, LANES)]
                    pltpu.async_copy(v_hbm.at[vidx], vbuf, sem).wait()
                    dstv = dst_blk[pl.ds(g * LANES, LANES)]
                    eglobv = jnp.broadcast_to(gl0, (LANES,)) + iota
                    okv = (eglobv >= lo) & (eglobv < hi)
                    rowlv = jnp.where(okv, dstv - clo, 0)
                    ams = []
                    for h in range(HEADS):
                        exv = jnp.exp(
                            alpha_blk[h, pl.ds(g * LANES, LANES)] - gm[h])
                        denv = plsc.load_gather(
                            den_local,
                            [rowlv, jnp.full((LANES,), h, jnp.int32)])
                        ams.append(jnp.where(okv, exv / denv, 0.0))
                    for r in range(LANES):
                        rowl = rowlv[r]
                        for h in range(HEADS):
                            a_sp = jnp.broadcast_to(ams[h][r], (LANES,))
                            for jj in range(HID // LANES):
                                col = h * HID + jj * LANES
                                plsc.addupdate(
                                    acc.at[rowl, pl.ds(col, LANES)],
                                    vbuf[r, pl.ds(col, LANES)] * a_sp)
                return carry2

            lax.fori_loop(0, BLK // LANES, p2_group, 0)
            return carry

        lax.fori_loop(0, cnblk, p2_block, 0)
        pltpu.sync_copy(acc, out_hbm.at[pl.ds(clo, R_CHUNK)])
        return carry0

    lax.fori_loop(0, D_SC // NS, chunk_body, 0)


def kernel(x, edge_index, params):
    src = edge_index[0].astype(jnp.int32)
    dst = edge_index[1].astype(jnp.int32)
    order = jnp.argsort(dst)
    dsts = dst[order]
    srcs = src[order]
    padlen = E_ALLOC - E
    dsts_p = jnp.concatenate([dsts, jnp.zeros((padlen,), jnp.int32)])
    srcs_p = jnp.concatenate([srcs, jnp.zeros((padlen,), jnp.int32)])

    mid = jnp.searchsorted(dsts, SC_HALF).astype(jnp.int32)
    scb = jnp.concatenate([
        jnp.zeros((1,), jnp.int32),
        mid[None],
        jnp.full((30,), E, jnp.int32),
    ])
    cgrid = jnp.arange(0, N_PAD + 1, R_CHUNK, dtype=jnp.int32)
    cb = jnp.searchsorted(dsts, cgrid).astype(jnp.int32)
    cb = jnp.concatenate([cb, jnp.full((144 - cb.shape[0],), E, jnp.int32)])

    b2 = lambda b: jnp.tile(b[None, :], (8, 1))

    h = _mm(x, params["proj_W"], b2(params["proj_b"]))
    for i in range(3):
        lp = params["layers"][i]
        q = _mm(h, lp["Wq"], b2(lp["bq"]))
        k = _mm(h, lp["Wk"], b2(lp["bk"]))
        v = _mm(h, lp["Wv"], b2(lp["bv"]))
        alpha, tmax = _phase_a(q, k, srcs_p, dsts_p)
        attn = _phase_bc(v, srcs_p, dsts_p, alpha, tmax, scb, cb)
        h = _mm(h, lp["Ws"], b2(lp["bs"]), c=attn[:N], relu=(i < 2))

    g1 = _mm(h, params["cls_W1"], b2(params["cls_b1"]), relu=True)
    w2 = jnp.pad(params["cls_W2"], ((0, 0), (0, 7)))
    bias2 = jnp.pad(params["cls_b2"], (0, 7))
    out = _mm(g1, w2, b2(bias2))
    return out[:, :121]


# speedup vs baseline: 7.2369x; 1.4274x over previous
import functools

import jax
import jax.numpy as jnp
from jax import lax
from jax.experimental import pallas as pl
from jax.experimental.pallas import tpu as pltpu
from jax.experimental.pallas import tpu_sc as plsc

N = 10000
E = 160000
HID = 256
HEADS = 4
FW = HID * HEADS

NC = 2
NS = 16
NW = NC * NS
LANES = 16

EPT = 5008
E_PAD = NW * EPT
E_ALLOC = E_PAD + 2048

R_CHUNK = 64
D_SC = 80
SC_HALF = R_CHUNK * D_SC
N_PAD = NC * SC_HALF
BLK = 512
NEG = -3.0e38

_mesh = plsc.VectorSubcoreMesh(core_axis_name="c", subcore_axis_name="s")
_sc_params = pltpu.CompilerParams(use_tc_tiling_on_sc=False,
                                  needs_layout_passes=False)


def _mm(a, w, b2d, c=None, relu=False):
    m, kdim = a.shape
    dout = w.shape[1]
    mt = 400
    nt = 256 if dout % 256 == 0 else 128

    def body(*refs):
        if c is None:
            a_ref, w_ref, b_ref, o_ref = refs
            acc = jnp.dot(a_ref[...], w_ref[...],
                          preferred_element_type=jnp.float32)
            acc = acc + b_ref[0][None, :]
        else:
            a_ref, w_ref, b_ref, c_ref, o_ref = refs
            acc = jnp.dot(a_ref[...], w_ref[...],
                          preferred_element_type=jnp.float32)
            acc = acc + b_ref[0][None, :] + c_ref[...]
        if relu:
            acc = jnp.maximum(acc, 0.0)
        o_ref[...] = acc

    in_specs = [
        pl.BlockSpec((mt, kdim), lambda i, j: (i, 0)),
        pl.BlockSpec((kdim, nt), lambda i, j: (0, j)),
        pl.BlockSpec((8, nt), lambda i, j: (0, j)),
    ]
    args = [a, w, b2d]
    if c is not None:
        in_specs.append(pl.BlockSpec((mt, nt), lambda i, j: (i, j)))
        args.append(c)
    return pl.pallas_call(
        body,
        grid=(m // mt, dout // nt),
        in_specs=in_specs,
        out_specs=pl.BlockSpec((mt, nt), lambda i, j: (i, j)),
        out_shape=jax.ShapeDtypeStruct((m, dout), jnp.float32),
    )(*args)


@functools.partial(
    pl.kernel,
    out_type=[
        jax.ShapeDtypeStruct((HEADS * E_ALLOC,), jnp.float32),
        jax.ShapeDtypeStruct((NW * HEADS * LANES,), jnp.float32),
    ],
    mesh=_mesh,
    scratch_types=[
        pltpu.VMEM((EPT,), jnp.int32),
        pltpu.VMEM((EPT,), jnp.int32),
        pltpu.VMEM((HEADS, EPT), jnp.float32),
        pltpu.VMEM((LANES, FW), jnp.float32),
        pltpu.VMEM((LANES, FW), jnp.float32),
        pltpu.VMEM((LANES, FW), jnp.float32),
        pltpu.VMEM((LANES, FW), jnp.float32),
        pltpu.VMEM((LANES,), jnp.int32),
        pltpu.VMEM((LANES,), jnp.int32),
        pltpu.VMEM((LANES,), jnp.int32),
        pltpu.VMEM((LANES,), jnp.int32),
        pltpu.VMEM((HEADS * LANES,), jnp.float32),
        pltpu.SemaphoreType.DMA,
        pltpu.SemaphoreType.DMA,
        pltpu.SemaphoreType.DMA,
        pltpu.SemaphoreType.DMA,
    ],
    compiler_params=_sc_params,
)
def _phase_a(q_hbm, k_hbm, src_hbm, dst_hbm, alpha_hbm, tmax_hbm,
             dst_t, src_t, alpha_st, qbuf, kbuf, qbuf1, kbuf1,
             qidx, kidx, qidx1, kidx1, rm,
             sem_q, sem_k, sem_q1, sem_k1):
    cid = lax.axis_index("c")
    sid = lax.axis_index("s")
    wid = sid * NC + cid
    base = pl.multiple_of(wid * EPT, 8)

    pltpu.sync_copy(dst_hbm.at[pl.ds(base, EPT)], dst_t)
    pltpu.sync_copy(src_hbm.at[pl.ds(base, EPT)], src_t)

    neg = jnp.full((LANES,), NEG, jnp.float32)
    for h in range(HEADS):
        rm[pl.ds(h * LANES, LANES)] = neg

    iota = lax.iota(jnp.int32, LANES)
    zero16 = jnp.zeros((LANES,), jnp.float32)

    def fire(g, qi, ki, qb, kb, sq, sk):
        qi[...] = dst_t[pl.ds(g * LANES, LANES)]
        ki[...] = src_t[pl.ds(g * LANES, LANES)]
        pltpu.async_copy(q_hbm.at[qi], qb, sq)
        pltpu.async_copy(k_hbm.at[ki], kb, sk)

    def waitbuf(qi, ki, qb, kb, sq, sk):
        pltpu.make_async_copy(q_hbm.at[qi], qb, sq).wait()
        pltpu.make_async_copy(k_hbm.at[ki], kb, sk).wait()

    def compute(g, qb, kb):
        def edge_dot(r, als):
            lane = iota == r
            new = []
            for h in range(HEADS):
                acc = (qb[r, pl.ds(h * HID, LANES)]
                       * kb[r, pl.ds(h * HID, LANES)])
                for j in range(1, HID // LANES):
                    col = h * HID + j * LANES
                    acc = acc + (qb[r, pl.ds(col, LANES)]
                                 * kb[r, pl.ds(col, LANES)])
                s = jnp.sum(acc) * 0.0625
                new.append(jnp.where(lane, s, als[h]))
            return tuple(new)

        als = lax.fori_loop(0, LANES, edge_dot, (zero16,) * HEADS)
        for h in range(HEADS):
            alpha_st[h, pl.ds(g * LANES, LANES)] = als[h]
            rm[pl.ds(h * LANES, LANES)] = jnp.maximum(
                rm[pl.ds(h * LANES, LANES)], als[h])

    ngroups = EPT // LANES
    fire(0, qidx, kidx, qbuf, kbuf, sem_q, sem_k)

    def pair(p, carry):
        g = p * 2
        fire(g + 1, qidx1, kidx1, qbuf1, kbuf1, sem_q1, sem_k1)
        waitbuf(qidx, kidx, qbuf, kbuf, sem_q, sem_k)
        compute(g, qbuf, kbuf)
        fire(g + 2, qidx, kidx, qbuf, kbuf, sem_q, sem_k)
        waitbuf(qidx1, kidx1, qbuf1, kbuf1, sem_q1, sem_k1)
        compute(g + 1, qbuf1, kbuf1)
        return carry

    lax.fori_loop(0, ngroups // 2, pair, 0)
    waitbuf(qidx, kidx, qbuf, kbuf, sem_q, sem_k)
    compute(ngroups - 1, qbuf, kbuf)

    for h in range(HEADS):
        pltpu.sync_copy(alpha_st.at[h],
                        alpha_hbm.at[pl.ds(pl.multiple_of(h * E_ALLOC + base, 8), EPT)])
    pltpu.sync_copy(rm, tmax_hbm.at[pl.ds(pl.multiple_of(wid * HEADS * LANES, 8), HEADS * LANES)])


@functools.partial(
    pl.kernel,
    out_type=jax.ShapeDtypeStruct((N_PAD, FW), jnp.float32),
    mesh=_mesh,
    scratch_types=[
        pltpu.VMEM((R_CHUNK, FW), jnp.float32),
        pltpu.VMEM((LANES, FW), jnp.float32),
        pltpu.VMEM((LANES, FW), jnp.float32),
        pltpu.VMEM((BLK,), jnp.int32),
        pltpu.VMEM((BLK,), jnp.int32),
        pltpu.VMEM((HEADS, BLK), jnp.float32),
        pltpu.VMEM((128, LANES), jnp.float32),
        pltpu.VMEM((128,), jnp.int32),
        pltpu.VMEM((LANES,), jnp.int32),
        pltpu.VMEM((LANES,), jnp.int32),
        pltpu.VMEM((32,), jnp.int32),
        pltpu.VMEM((HEADS, LANES), jnp.float32),
        pltpu.VMEM((R_CHUNK, LANES), jnp.float32),
        pltpu.VMEM((NW * HEADS * LANES,), jnp.float32),
        pltpu.VMEM((32,), jnp.int32),
        pltpu.VMEM((176,), jnp.int32),
        pltpu.VMEM_SHARED((SC_HALF, LANES), jnp.float32),
        pltpu.SemaphoreType.DMA,
        pltpu.SemaphoreType.DMA,
    ],
    compiler_params=_sc_params,
)
def _phase_bc(v_hbm, src_hbm, dst_hbm, alpha_hbm, tmax_hbm, scb_hbm, cb_hbm,
              out_hbm,
              acc, vbuf, vbuf1, dst_blk, src_blk, alpha_blk, exrow, exidx,
              vidx, vidx1, rlbuf, amsbuf, den_local, tmax_v, scb_v, cb_v,
              dshared, sem, sem1):
    cid = lax.axis_index("c")
    sid = lax.axis_index("s")

    pltpu.sync_copy(tmax_hbm, tmax_v)
    pltpu.sync_copy(scb_hbm, scb_v)
    pltpu.sync_copy(cb_hbm, cb_v)

    gm = []
    for h in range(HEADS):
        m = tmax_v[pl.ds(h * LANES, LANES)]
        for w in range(1, NW):
            m = jnp.maximum(
                m, tmax_v[pl.ds((w * HEADS + h) * LANES, LANES)])
        gm.append(jnp.broadcast_to(jnp.max(m), (LANES,)))

    iota = lax.iota(jnp.int32, LANES)
    zero16 = jnp.zeros((LANES,), jnp.float32)

    def _sread(ref, idx):
        return ref[pl.ds(idx, LANES)][0]

    for r in range(R_CHUNK):
        den_local[r, :] = zero16
    @pl.when(sid == 0)
    def _zero_shared():
        def zs(i, carry):
            pltpu.sync_copy(den_local, dshared.at[pl.ds(i * R_CHUNK, R_CHUNK)])
            return carry
        lax.fori_loop(0, D_SC, zs, 0)
    plsc.subcore_barrier()

    for r in range(128):
        exrow[r, :] = zero16
    izero16 = jnp.zeros((LANES,), jnp.int32)
    rlbuf[pl.ds(0, LANES)] = izero16
    rlbuf[pl.ds(LANES, LANES)] = izero16

    lo_sc = _sread(scb_v, cid)
    hi_sc = _sread(scb_v, cid + 1)
    per = (hi_sc - lo_sc + NS - 1) >> 4
    mylo = lo_sc + sid * per
    myhi = jnp.minimum(mylo + per, hi_sc)
    blk0 = mylo & ~7
    nblk = jnp.maximum(0, (myhi - blk0 + BLK - 1) >> 9)
    dbase = cid * SC_HALF

    def p1_block(b, carry):
        estart = pl.multiple_of(blk0 + b * BLK, 8)
        pltpu.sync_copy(dst_hbm.at[pl.ds(estart, BLK)], dst_blk)
        for h in range(HEADS):
            pltpu.sync_copy(alpha_hbm.at[pl.ds(pl.multiple_of(h * E_ALLOC + estart, 8), BLK)],
                            alpha_blk.at[h])

        def p1_group(g, carry2):
            gb = lax.rem(g, 8)
            dstv = dst_blk[pl.ds(g * LANES, LANES)]
            eglob = jnp.broadcast_to(estart + g * LANES, (LANES,)) + iota
            valid = (eglob >= mylo) & (eglob < myhi)
            idxv = jnp.where(valid, dstv - dbase, 0)
            exs = []
            for h in range(HEADS):
                av = alpha_blk[h, pl.ds(g * LANES, LANES)]
                ex = jnp.exp(av - gm[h])
                exs.append(jnp.where(valid, ex, 0.0))
            gbase = gb * LANES
            for r in range(LANES):
                rowv = jnp.where(iota == 0, exs[0][r], 0.0)
                for h in range(1, HEADS):
                    rowv = jnp.where(iota == h, exs[h][r], rowv)
                exrow[gbase + r, :] = rowv
            exidx[pl.ds(gbase, LANES)] = idxv

            @pl.when(gb == 7)
            def _flush():
                pltpu.sync_copy(exrow, dshared.at[exidx], add=True)
            return carry2

        lax.fori_loop(0, BLK // LANES, p1_group, 0)
        return carry

    lax.fori_loop(0, nblk, p1_block, 0)
    plsc.subcore_barrier()

    def chunk_body(i, carry0):
        c_sc = sid + NS * i
        c_glob = cid * D_SC + c_sc
        clo = c_glob * R_CHUNK

        pltpu.sync_copy(dshared.at[pl.ds(c_sc * R_CHUNK, R_CHUNK)], den_local)

        def zacc(r, carry):
            for j in range(FW // LANES):
                acc[r, pl.ds(j * LANES, LANES)] = zero16
            return carry
        lax.fori_loop(0, R_CHUNK, zacc, 0)

        lo = _sread(cb_v, c_glob)
        hi = _sread(cb_v, c_glob + 1)
        cblk0 = lo & ~7
        cnblk = jnp.maximum(0, (hi - cblk0 + BLK - 1) >> 9)

        def p2_block(b, carry):
            estart = pl.multiple_of(cblk0 + b * BLK, 8)
            pltpu.sync_copy(dst_hbm.at[pl.ds(estart, BLK)], dst_blk)
            pltpu.sync_copy(src_hbm.at[pl.ds(estart, BLK)], src_blk)
            for h in range(HEADS):
                pltpu.sync_copy(alpha_hbm.at[pl.ds(pl.multiple_of(h * E_ALLOC + estart, 8), BLK)],
                                alpha_blk.at[h])

            g1 = jnp.minimum(BLK // LANES,
                             jnp.maximum(0, (hi - estart + LANES - 1) >> 4))

            def fire_v(g, vi, vb, sm):
                vi[...] = src_blk[pl.ds(g * LANES, LANES)]
                pltpu.async_copy(v_hbm.at[vi], vb, sm)

            def compute_v(g, vb):
                gl0 = estart + g * LANES
                dstv = dst_blk[pl.ds(g * LANES, LANES)]
                eglobv = jnp.broadcast_to(gl0, (LANES,)) + iota
                okv = (eglobv >= lo) & (eglobv < hi)
                rowlv = jnp.where(okv, dstv - clo, 0)
                rlbuf[pl.ds(0, LANES)] = rowlv
                for h in range(HEADS):
                    exv = jnp.exp(
                        alpha_blk[h, pl.ds(g * LANES, LANES)] - gm[h])
                    denv = plsc.load_gather(
                        den_local,
                        [rowlv, jnp.full((LANES,), h, jnp.int32)])
                    amsbuf[h, :] = jnp.where(okv, exv / denv, 0.0)

                def edge(r, c3):
                    rowl = rlbuf[pl.ds(r, LANES)][0]
                    rsp = jnp.full((LANES,), r, jnp.int32)
                    for h in range(HEADS):
                        a_sp = plsc.load_gather(amsbuf.at[h], [rsp])
                        for jj in range(HID // LANES):
                            col = h * HID + jj * LANES
                            plsc.addupdate(
                                acc.at[rowl, pl.ds(col, LANES)],
                                vb[r, pl.ds(col, LANES)] * a_sp)
                    return c3
                lax.fori_loop(0, LANES, edge, 0)

            fire_v(0, vidx, vbuf, sem)

            def vpair(p, c2):
                g = 2 * p

                @pl.when(g + 1 < g1)
                def _f1():
                    fire_v(g + 1, vidx1, vbuf1, sem1)
                pltpu.make_async_copy(v_hbm.at[vidx], vbuf, sem).wait()
                compute_v(g, vbuf)

                @pl.when(g + 2 < g1)
                def _f2():
                    fire_v(g + 2, vidx, vbuf, sem)

                @pl.when(g + 1 < g1)
                def _c1():
                    pltpu.make_async_copy(v_hbm.at[vidx1], vbuf1, sem1).wait()
                    compute_v(g + 1, vbuf1)
                return c2

            lax.fori_loop(0, (g1 + 1) >> 1, vpair, 0)
            return carry

        lax.fori_loop(0, cnblk, p2_block, 0)
        pltpu.sync_copy(acc, out_hbm.at[pl.ds(clo, R_CHUNK)])
        return carry0

    lax.fori_loop(0, D_SC // NS, chunk_body, 0)


def kernel(x, edge_index, params):
    src = edge_index[0].astype(jnp.int32)
    dst = edge_index[1].astype(jnp.int32)
    order = jnp.argsort(dst)
    dsts = dst[order]
    srcs = src[order]
    padlen = E_ALLOC - E
    dsts_p = jnp.concatenate([dsts, jnp.zeros((padlen,), jnp.int32)])
    srcs_p = jnp.concatenate([srcs, jnp.zeros((padlen,), jnp.int32)])

    mid = jnp.searchsorted(dsts, SC_HALF).astype(jnp.int32)
    scb = jnp.concatenate([
        jnp.zeros((1,), jnp.int32),
        mid[None],
        jnp.full((30,), E, jnp.int32),
    ])
    cgrid = jnp.arange(0, N_PAD + 1, R_CHUNK, dtype=jnp.int32)
    cb = jnp.searchsorted(dsts, cgrid).astype(jnp.int32)
    cb = jnp.concatenate([cb, jnp.full((176 - cb.shape[0],), E, jnp.int32)])

    b2 = lambda b: jnp.tile(b[None, :], (8, 1))

    h = _mm(x, params["proj_W"], b2(params["proj_b"]))
    for i in range(3):
        lp = params["layers"][i]
        q = _mm(h, lp["Wq"], b2(lp["bq"]))
        k = _mm(h, lp["Wk"], b2(lp["bk"]))
        v = _mm(h, lp["Wv"], b2(lp["bv"]))
        alpha, tmax = _phase_a(q, k, srcs_p, dsts_p)
        attn = _phase_bc(v, srcs_p, dsts_p, alpha, tmax, scb, cb)
        h = _mm(h, lp["Ws"], b2(lp["bs"]), c=attn[:N], relu=(i < 2))

    g1 = _mm(h, params["cls_W1"], b2(params["cls_b1"]), relu=True)
    w2 = jnp.pad(params["cls_W2"], ((0, 0), (0, 7)))
    bias2 = jnp.pad(params["cls_b2"], (0, 7))
    out = _mm(g1, w2, b2(bias2))
    return out[:, :121]
